# Initial kernel scaffold; baseline (speedup 1.0000x reference)
import jax
import jax.numpy as jnp
from jax.experimental import pallas as pl


def kernel(mention_hidden_state, entity_hidden_state, sent_hidden_state, edge_index, type_emb, ln_gamma, ln_beta, W0, b0, W1, b1, fc_W, fc_b):
    raise NotImplementedError("write your pallas kernel here")


# speedup vs baseline: 4.0581x; 4.0581x over previous
import functools

import jax
import jax.numpy as jnp
from jax import lax
from jax.experimental import pallas as pl
from jax.experimental.pallas import tpu as pltpu
from jax.experimental.pallas import tpu_sc as plsc

N_NODES = 10000
NPAD = 10240
D = 128
DH2 = 64
NC, NS, L = 2, 16, 16
NW = NC * NS
K = 128
EPT = 20480
CH = EPT // K
EPAD = NS * EPT
CHD = EPAD // (NW * K)
RPT = NPAD // NS
PAD_NODE = N_NODES + 64

_mesh = plsc.VectorSubcoreMesh(core_axis_name="c", subcore_axis_name="s",
                               num_cores=NC)


@functools.partial(
    pl.kernel,
    out_type=jax.ShapeDtypeStruct((NC, 2, NPAD, 32), jnp.float32),
    mesh=_mesh,
    scratch_types=[
        pltpu.VMEM((K,), jnp.int32),
        pltpu.VMEM((K,), jnp.int32),
        pltpu.VMEM((K, 32), jnp.float32),
        pltpu.VMEM_SHARED((NPAD, 32), jnp.float32),
        pltpu.VMEM_SHARED((NPAD, 32), jnp.float32),
    ],
    compiler_params=pltpu.CompilerParams(use_tc_tiling_on_sc=False),
)
def _sc_degrees(src_hbm, dst_hbm, ones_hbm, zeros_hbm, out_hbm,
                sidx, didx, ones_v, dgo_s, dgi_s):
    c = lax.axis_index("c")
    s = lax.axis_index("s")
    wid = c * NS + s
    pltpu.sync_copy(ones_hbm, ones_v)
    pltpu.sync_copy(zeros_hbm.at[pl.ds(s * RPT, RPT)], dgo_s.at[pl.ds(s * RPT, RPT)])
    pltpu.sync_copy(zeros_hbm.at[pl.ds(s * RPT, RPT)], dgi_s.at[pl.ds(s * RPT, RPT)])
    plsc.subcore_barrier()

    def eloop(t, carry):
        pltpu.sync_copy(src_hbm.at[wid, t], sidx)
        pltpu.sync_copy(ones_v, dgo_s.at[sidx], add=True)
        pltpu.sync_copy(dst_hbm.at[wid, t], didx)
        pltpu.sync_copy(ones_v, dgi_s.at[didx], add=True)
        return carry

    lax.fori_loop(0, CHD, eloop, 0)
    plsc.subcore_barrier()
    pltpu.sync_copy(dgo_s.at[pl.ds(s * RPT, RPT)],
                    out_hbm.at[c, 0, pl.ds(s * RPT, RPT)])
    pltpu.sync_copy(dgi_s.at[pl.ds(s * RPT, RPT)],
                    out_hbm.at[c, 1, pl.ds(s * RPT, RPT)])


@functools.partial(
    pl.kernel,
    out_type=jax.ShapeDtypeStruct((NC, NPAD, DH2), jnp.float32),
    mesh=_mesh,
    scratch_types=[
        pltpu.VMEM((CH, K), jnp.int32),
        pltpu.VMEM((K,), jnp.int32),
        pltpu.VMEM((2, K, DH2), jnp.float32),
        pltpu.VMEM_SHARED((NPAD, DH2), jnp.float32),
        pltpu.SemaphoreType.DMA,
        pltpu.SemaphoreType.DMA,
    ],
    compiler_params=pltpu.CompilerParams(use_tc_tiling_on_sc=False),
)
def _sc_gather_scatter(h_hbm, src_hbm, dst_hbm, zeros_hbm, out_hbm,
                       sidx, didx, rows_v, agg_s, sem0, sem1):
    c = lax.axis_index("c")
    s = lax.axis_index("s")
    pltpu.sync_copy(zeros_hbm.at[pl.ds(s * RPT, RPT)], agg_s.at[pl.ds(s * RPT, RPT)])
    pltpu.sync_copy(src_hbm.at[c, s], sidx)
    plsc.subcore_barrier()

    sems = (sem0, sem1)
    pltpu.async_copy(h_hbm.at[sidx.at[0]], rows_v.at[0], sem0)

    def eloop(t2, carry):
        for k in range(2):
            t = 2 * t2 + k
            buf = rows_v.at[k]
            obuf = rows_v.at[1 - k]
            pltpu.sync_copy(dst_hbm.at[s, t], didx)
            pltpu.make_async_copy(h_hbm.at[sidx.at[t]], buf, sems[k]).wait()

            @pl.when(t + 1 < CH)
            def _():
                pltpu.async_copy(h_hbm.at[sidx.at[t + 1]], obuf, sems[1 - k])

            pltpu.sync_copy(buf, agg_s.at[didx], add=True)
        return carry

    lax.fori_loop(0, CH // 2, eloop, 0)
    plsc.subcore_barrier()
    pltpu.sync_copy(agg_s.at[pl.ds(s * RPT, RPT)],
                    out_hbm.at[c, pl.ds(s * RPT, RPT)])


def _norm_body(dp_ref, o_ref):
    do = dp_ref[0, 0] + dp_ref[1, 0]
    di = dp_ref[0, 1] + dp_ref[1, 1]
    o_ref[0] = lax.rsqrt(jnp.maximum(do, 1.0))
    o_ref[1] = lax.rsqrt(jnp.maximum(di, 1.0))


def _ln_scale_body(node_ref, g_ref, b_ref, ns_ref, o_ref):
    xb = node_ref[...]
    m = jnp.mean(xb, axis=1, keepdims=True)
    v = jnp.mean((xb - m) * (xb - m), axis=1, keepdims=True)
    y = ((xb - m) * lax.rsqrt(v + 1e-5) * g_ref[...] + b_ref[...]) * ns_ref[...]
    o_ref[0] = y[:, :DH2]
    o_ref[1] = y[:, DH2:]


def _layer_body(agg_ref, nd_ref, ns_ref, w_ref, b_ref, o_ref):
    a = jnp.concatenate([agg_ref[0], agg_ref[1]], axis=1) * nd_ref[...]
    z = jnp.dot(a, w_ref[...], preferred_element_type=jnp.float32) + b_ref[...]
    z = jnp.where(z >= 0, z, 0.01 * z)
    y = z * ns_ref[...]
    o_ref[0] = y[:, :DH2]
    o_ref[1] = y[:, DH2:]


def _final_body(agg_ref, nd_ref, w_ref, b_ref, fw_ref, fb_ref, o_ref):
    a = jnp.concatenate([agg_ref[0], agg_ref[1]], axis=1) * nd_ref[...]
    z = jnp.dot(a, w_ref[...], preferred_element_type=jnp.float32) + b_ref[...]
    z = jnp.where(z >= 0, z, 0.01 * z)
    o_ref[...] = jnp.dot(z, fw_ref[...], preferred_element_type=jnp.float32) + fb_ref[...]


_R = 1024


def _tc_norms(degp4):
    return pl.pallas_call(
        _norm_body,
        out_shape=jax.ShapeDtypeStruct((2, NPAD // 128, 128), jnp.float32),
    )(degp4)


def _tc_ln_scale(node, g2, b2, ns):
    return pl.pallas_call(
        _ln_scale_body,
        grid=(NPAD // _R,),
        in_specs=[
            pl.BlockSpec((_R, D), lambda i: (i, 0)),
            pl.BlockSpec((1, D), lambda i: (0, 0)),
            pl.BlockSpec((1, D), lambda i: (0, 0)),
            pl.BlockSpec((_R, 1), lambda i: (i, 0)),
        ],
        out_specs=pl.BlockSpec((2, _R, DH2), lambda i: (0, i, 0)),
        out_shape=jax.ShapeDtypeStruct((2, NPAD, DH2), jnp.float32),
    )(node, g2, b2, ns)


def _tc_layer(agg, nd, ns, W, b2):
    return pl.pallas_call(
        _layer_body,
        grid=(NPAD // _R,),
        in_specs=[
            pl.BlockSpec((2, _R, DH2), lambda i: (0, i, 0)),
            pl.BlockSpec((_R, 1), lambda i: (i, 0)),
            pl.BlockSpec((_R, 1), lambda i: (i, 0)),
            pl.BlockSpec((D, D), lambda i: (0, 0)),
            pl.BlockSpec((1, D), lambda i: (0, 0)),
        ],
        out_specs=pl.BlockSpec((2, _R, DH2), lambda i: (0, i, 0)),
        out_shape=jax.ShapeDtypeStruct((2, NPAD, DH2), jnp.float32),
    )(agg, nd, ns, W, b2)


def _tc_final(agg_e, nd_e, W, b2, fW, fb2):
    ne = agg_e.shape[1]
    return pl.pallas_call(
        _final_body,
        out_shape=jax.ShapeDtypeStruct((ne, D), jnp.float32),
    )(agg_e, nd_e, W, b2, fW, fb2)


def kernel(mention_hidden_state, entity_hidden_state, sent_hidden_state,
           edge_index, type_emb, ln_gamma, ln_beta, W0, b0, W1, b1, fc_W, fc_b):
    B, NM, DHS = mention_hidden_state.shape
    NE = entity_hidden_state.shape[1]
    NSn = sent_hidden_state.shape[1]
    DT = type_emb.shape[1]
    num_node = NM + NE + NSn
    E = edge_index.shape[1]

    m = jnp.concatenate(
        [mention_hidden_state,
         jnp.broadcast_to(type_emb[0].reshape(1, 1, DT), (B, NM, DT))], axis=2)
    e = jnp.concatenate(
        [entity_hidden_state,
         jnp.broadcast_to(type_emb[1].reshape(1, 1, DT), (B, NE, DT))], axis=2)
    sn = jnp.concatenate(
        [sent_hidden_state,
         jnp.broadcast_to(type_emb[2].reshape(1, 1, DT), (B, NSn, DT))], axis=2)
    node = jnp.concatenate((m, e, sn), axis=1).reshape(B * num_node, D)
    node = jnp.concatenate(
        [node, jnp.zeros((NPAD - B * num_node, D), jnp.float32)], axis=0)

    src = edge_index[0].astype(jnp.int32)
    dst = edge_index[1].astype(jnp.int32)
    pad_idx = jnp.full((EPAD - E,), PAD_NODE, jnp.int32)
    srcp = jnp.concatenate([src, pad_idx])
    dstp = jnp.concatenate([dst, pad_idx])
    src3d = srcp.reshape(NW, CHD, K)
    dst3d = dstp.reshape(NW, CHD, K)
    src4 = jnp.stack([srcp, srcp + NPAD]).reshape(NC, NS, CH, K)
    dst3 = dstp.reshape(NS, CH, K)

    ones32 = jnp.ones((K, 32), jnp.float32)
    zeros32 = jnp.zeros((NPAD, 32), jnp.float32)
    zeros64 = jnp.zeros((NPAD, DH2), jnp.float32)

    degp = _sc_degrees(src3d, dst3d, ones32, zeros32)
    degp4 = degp[:, :, :, 0].reshape(2, 2, NPAD // 128, 128)
    norms = _tc_norms(degp4)
    ns = norms[0].reshape(NPAD, 1)
    nd = norms[1].reshape(NPAD, 1)

    g2 = ln_gamma.reshape(1, D)
    be2 = ln_beta.reshape(1, D)
    h0 = _tc_ln_scale(node, g2, be2, ns)

    agg1 = _sc_gather_scatter(h0.reshape(2 * NPAD, DH2), src4, dst3, zeros64)
    h1 = _tc_layer(agg1, nd, ns, W0, b0.reshape(1, D))
    agg2 = _sc_gather_scatter(h1.reshape(2 * NPAD, DH2), src4, dst3, zeros64)

    agg2e = (agg2[:, :B * num_node]
             .reshape(2, B, num_node, DH2)[:, :, NM:NM + NE]
             .reshape(2, B * NE, DH2))
    nde = (nd[:B * num_node]
           .reshape(B, num_node, 1)[:, NM:NM + NE]
           .reshape(B * NE, 1))
    out = _tc_final(agg2e, nde, W1, b1.reshape(1, D),
                    fc_W, fc_b.reshape(1, D))
    return out.reshape(B, NE, D)


# speedup vs baseline: 5.6019x; 1.3804x over previous
import functools

import jax
import jax.numpy as jnp
from jax import lax
from jax.experimental import pallas as pl
from jax.experimental.pallas import tpu as pltpu
from jax.experimental.pallas import tpu_sc as plsc

N_NODES = 10000
NPAD = 10240
D = 128
DH2 = 64
NC, NS, L = 2, 16, 16
NW = NC * NS
K = 128
EPT = 20480
CH = EPT // K
EPAD = NS * EPT
CHD = EPAD // (NW * K)
RPT = NPAD // NS
PAD_NODE = N_NODES + 64

_mesh = plsc.VectorSubcoreMesh(core_axis_name="c", subcore_axis_name="s",
                               num_cores=NC)


@functools.partial(
    pl.kernel,
    out_type=jax.ShapeDtypeStruct((NC, 2, NPAD, 16), jnp.float32),
    mesh=_mesh,
    scratch_types=[
        pltpu.VMEM((CHD, K), jnp.int32),
        pltpu.VMEM((CHD, K), jnp.int32),
        pltpu.VMEM((K, 16), jnp.float32),
        pltpu.VMEM_SHARED((NPAD, 16), jnp.float32),
        pltpu.VMEM_SHARED((NPAD, 16), jnp.float32),
        pltpu.SemaphoreType.DMA,
        pltpu.SemaphoreType.DMA,
        pltpu.SemaphoreType.DMA,
        pltpu.SemaphoreType.DMA,
    ],
    compiler_params=pltpu.CompilerParams(use_tc_tiling_on_sc=False),
)
def _sc_degrees(src_hbm, dst_hbm, ones_hbm, zeros_hbm, out_hbm,
                sidx, didx, ones_v, dgo_s, dgi_s, ss0, ss1, ss2, ss3):
    c = lax.axis_index("c")
    s = lax.axis_index("s")
    wid = c * NS + s
    pltpu.sync_copy(ones_hbm, ones_v)
    pltpu.sync_copy(zeros_hbm.at[pl.ds(s * RPT, RPT)], dgo_s.at[pl.ds(s * RPT, RPT)])
    pltpu.sync_copy(zeros_hbm.at[pl.ds(s * RPT, RPT)], dgi_s.at[pl.ds(s * RPT, RPT)])
    pltpu.sync_copy(src_hbm.at[wid], sidx)
    pltpu.sync_copy(dst_hbm.at[wid], didx)
    plsc.subcore_barrier()

    sss = (ss0, ss1, ss2, ss3)

    def eloop(t4, carry):
        for k in range(4):
            t = 4 * t4 + k

            @pl.when(t >= 4)
            def _():
                pltpu.make_async_copy(ones_v, dgo_s.at[sidx.at[0]], sss[k]).wait()
                pltpu.make_async_copy(ones_v, dgi_s.at[didx.at[0]], sss[k]).wait()

            pltpu.async_copy(ones_v, dgo_s.at[sidx.at[t]], sss[k], add=True)
            pltpu.async_copy(ones_v, dgi_s.at[didx.at[t]], sss[k], add=True)
        return carry

    lax.fori_loop(0, CHD // 4, eloop, 0)
    for k in range(4):
        pltpu.make_async_copy(ones_v, dgo_s.at[sidx.at[0]], sss[k]).wait()
        pltpu.make_async_copy(ones_v, dgi_s.at[didx.at[0]], sss[k]).wait()
    plsc.subcore_barrier()
    pltpu.sync_copy(dgo_s.at[pl.ds(s * RPT, RPT)],
                    out_hbm.at[c, 0, pl.ds(s * RPT, RPT)])
    pltpu.sync_copy(dgi_s.at[pl.ds(s * RPT, RPT)],
                    out_hbm.at[c, 1, pl.ds(s * RPT, RPT)])


@functools.partial(
    pl.kernel,
    out_type=jax.ShapeDtypeStruct((NC, NPAD, DH2), jnp.float32),
    mesh=_mesh,
    scratch_types=[
        pltpu.VMEM((CH, K), jnp.int32),
        pltpu.VMEM((CH, K), jnp.int32),
        pltpu.VMEM((4, K, DH2), jnp.float32),
        pltpu.VMEM_SHARED((NPAD, DH2), jnp.float32),
        pltpu.SemaphoreType.DMA,
        pltpu.SemaphoreType.DMA,
        pltpu.SemaphoreType.DMA,
        pltpu.SemaphoreType.DMA,
        pltpu.SemaphoreType.DMA,
        pltpu.SemaphoreType.DMA,
        pltpu.SemaphoreType.DMA,
        pltpu.SemaphoreType.DMA,
    ],
    compiler_params=pltpu.CompilerParams(use_tc_tiling_on_sc=False),
)
def _sc_gather_scatter(h_hbm, src_hbm, dst_hbm, zeros_hbm, out_hbm,
                       sidx, didx, rows_v, agg_s,
                       gs0, gs1, gs2, gs3, ss0, ss1, ss2, ss3):
    c = lax.axis_index("c")
    s = lax.axis_index("s")
    pltpu.sync_copy(zeros_hbm.at[pl.ds(s * RPT, RPT)], agg_s.at[pl.ds(s * RPT, RPT)])
    pltpu.sync_copy(src_hbm.at[c, s], sidx)
    pltpu.sync_copy(dst_hbm.at[s], didx)
    plsc.subcore_barrier()

    gss = (gs0, gs1, gs2, gs3)
    sss = (ss0, ss1, ss2, ss3)
    pltpu.async_copy(h_hbm.at[sidx.at[0]], rows_v.at[0], gs0)
    pltpu.async_copy(h_hbm.at[sidx.at[1]], rows_v.at[1], gs1)

    def eloop(t4, carry):
        for k in range(4):
            t = 4 * t4 + k
            k2 = (k + 2) % 4
            pltpu.make_async_copy(h_hbm.at[sidx.at[t]], rows_v.at[k], gss[k]).wait()

            @pl.when(t + 2 < CH)
            def _():
                @pl.when(t >= 2)
                def _():
                    pltpu.make_async_copy(
                        rows_v.at[k2], agg_s.at[didx.at[0]], sss[k2]).wait()

                pltpu.async_copy(h_hbm.at[sidx.at[t + 2]], rows_v.at[k2], gss[k2])

            pltpu.async_copy(rows_v.at[k], agg_s.at[didx.at[t]], sss[k], add=True)
        return carry

    lax.fori_loop(0, CH // 4, eloop, 0)
    for j in range(4):
        pltpu.make_async_copy(rows_v.at[j], agg_s.at[didx.at[0]], sss[j]).wait()
    plsc.subcore_barrier()
    pltpu.sync_copy(agg_s.at[pl.ds(s * RPT, RPT)],
                    out_hbm.at[c, pl.ds(s * RPT, RPT)])


def _norm_body(dp_ref, o_ref):
    do = dp_ref[0, 0] + dp_ref[1, 0]
    di = dp_ref[0, 1] + dp_ref[1, 1]
    o_ref[0] = lax.rsqrt(jnp.maximum(do, 1.0))
    o_ref[1] = lax.rsqrt(jnp.maximum(di, 1.0))


def _ln_scale_body(node_ref, g_ref, b_ref, ns_ref, o_ref):
    xb = node_ref[...]
    m = jnp.mean(xb, axis=1, keepdims=True)
    v = jnp.mean((xb - m) * (xb - m), axis=1, keepdims=True)
    y = ((xb - m) * lax.rsqrt(v + 1e-5) * g_ref[...] + b_ref[...]) * ns_ref[...]
    o_ref[0] = y[:, :DH2]
    o_ref[1] = y[:, DH2:]


def _layer_body(agg_ref, nd_ref, ns_ref, w_ref, b_ref, o_ref):
    a = jnp.concatenate([agg_ref[0], agg_ref[1]], axis=1) * nd_ref[...]
    z = jnp.dot(a, w_ref[...], preferred_element_type=jnp.float32) + b_ref[...]
    z = jnp.where(z >= 0, z, 0.01 * z)
    y = z * ns_ref[...]
    o_ref[0] = y[:, :DH2]
    o_ref[1] = y[:, DH2:]


def _final_body(agg_ref, nd_ref, w_ref, b_ref, fw_ref, fb_ref, o_ref):
    a = jnp.concatenate([agg_ref[0], agg_ref[1]], axis=1) * nd_ref[...]
    z = jnp.dot(a, w_ref[...], preferred_element_type=jnp.float32) + b_ref[...]
    z = jnp.where(z >= 0, z, 0.01 * z)
    o_ref[...] = jnp.dot(z, fw_ref[...], preferred_element_type=jnp.float32) + fb_ref[...]


_R = 1024


def _tc_norms(degp4):
    return pl.pallas_call(
        _norm_body,
        out_shape=jax.ShapeDtypeStruct((2, NPAD // 128, 128), jnp.float32),
    )(degp4)


def _tc_ln_scale(node, g2, b2, ns):
    return pl.pallas_call(
        _ln_scale_body,
        grid=(NPAD // _R,),
        in_specs=[
            pl.BlockSpec((_R, D), lambda i: (i, 0)),
            pl.BlockSpec((1, D), lambda i: (0, 0)),
            pl.BlockSpec((1, D), lambda i: (0, 0)),
            pl.BlockSpec((_R, 1), lambda i: (i, 0)),
        ],
        out_specs=pl.BlockSpec((2, _R, DH2), lambda i: (0, i, 0)),
        out_shape=jax.ShapeDtypeStruct((2, NPAD, DH2), jnp.float32),
    )(node, g2, b2, ns)


def _tc_layer(agg, nd, ns, W, b2):
    return pl.pallas_call(
        _layer_body,
        grid=(NPAD // _R,),
        in_specs=[
            pl.BlockSpec((2, _R, DH2), lambda i: (0, i, 0)),
            pl.BlockSpec((_R, 1), lambda i: (i, 0)),
            pl.BlockSpec((_R, 1), lambda i: (i, 0)),
            pl.BlockSpec((D, D), lambda i: (0, 0)),
            pl.BlockSpec((1, D), lambda i: (0, 0)),
        ],
        out_specs=pl.BlockSpec((2, _R, DH2), lambda i: (0, i, 0)),
        out_shape=jax.ShapeDtypeStruct((2, NPAD, DH2), jnp.float32),
    )(agg, nd, ns, W, b2)


def _tc_final(agg_e, nd_e, W, b2, fW, fb2):
    ne = agg_e.shape[1]
    return pl.pallas_call(
        _final_body,
        out_shape=jax.ShapeDtypeStruct((ne, D), jnp.float32),
    )(agg_e, nd_e, W, b2, fW, fb2)


def kernel(mention_hidden_state, entity_hidden_state, sent_hidden_state,
           edge_index, type_emb, ln_gamma, ln_beta, W0, b0, W1, b1, fc_W, fc_b):
    B, NM, DHS = mention_hidden_state.shape
    NE = entity_hidden_state.shape[1]
    NSn = sent_hidden_state.shape[1]
    DT = type_emb.shape[1]
    num_node = NM + NE + NSn
    E = edge_index.shape[1]

    m = jnp.concatenate(
        [mention_hidden_state,
         jnp.broadcast_to(type_emb[0].reshape(1, 1, DT), (B, NM, DT))], axis=2)
    e = jnp.concatenate(
        [entity_hidden_state,
         jnp.broadcast_to(type_emb[1].reshape(1, 1, DT), (B, NE, DT))], axis=2)
    sn = jnp.concatenate(
        [sent_hidden_state,
         jnp.broadcast_to(type_emb[2].reshape(1, 1, DT), (B, NSn, DT))], axis=2)
    node = jnp.concatenate((m, e, sn), axis=1).reshape(B * num_node, D)
    node = jnp.concatenate(
        [node, jnp.zeros((NPAD - B * num_node, D), jnp.float32)], axis=0)

    src = edge_index[0].astype(jnp.int32)
    dst = edge_index[1].astype(jnp.int32)
    pad_idx = jnp.full((EPAD - E,), PAD_NODE, jnp.int32)
    srcp = jnp.concatenate([src, pad_idx])
    dstp = jnp.concatenate([dst, pad_idx])
    src3d = srcp.reshape(NW, CHD, K)
    dst3d = dstp.reshape(NW, CHD, K)
    src4 = jnp.stack([srcp, srcp + NPAD]).reshape(NC, NS, CH, K)
    dst3 = dstp.reshape(NS, CH, K)

    ones16 = jnp.ones((K, 16), jnp.float32)
    zeros16 = jnp.zeros((NPAD, 16), jnp.float32)
    zeros64 = jnp.zeros((NPAD, DH2), jnp.float32)

    degp = _sc_degrees(src3d, dst3d, ones16, zeros16)
    degp4 = degp[:, :, :, 0].reshape(2, 2, NPAD // 128, 128)
    norms = _tc_norms(degp4)
    ns = norms[0].reshape(NPAD, 1)
    nd = norms[1].reshape(NPAD, 1)

    g2 = ln_gamma.reshape(1, D)
    be2 = ln_beta.reshape(1, D)
    h0 = _tc_ln_scale(node, g2, be2, ns)

    agg1 = _sc_gather_scatter(h0.reshape(2 * NPAD, DH2), src4, dst3, zeros64)
    h1 = _tc_layer(agg1, nd, ns, W0, b0.reshape(1, D))
    agg2 = _sc_gather_scatter(h1.reshape(2 * NPAD, DH2), src4, dst3, zeros64)

    agg2e = (agg2[:, :B * num_node]
             .reshape(2, B, num_node, DH2)[:, :, NM:NM + NE]
             .reshape(2, B * NE, DH2))
    nde = (nd[:B * num_node]
           .reshape(B, num_node, 1)[:, NM:NM + NE]
           .reshape(B * NE, 1))
    out = _tc_final(agg2e, nde, W1, b1.reshape(1, D),
                    fc_W, fc_b.reshape(1, D))
    return out.reshape(B, NE, D)
